# Initial kernel scaffold; baseline (speedup 1.0000x reference)
import jax
import jax.numpy as jnp
from jax.experimental import pallas as pl


def kernel(x, Wq, Wkv, Wexp):
    raise NotImplementedError("write your pallas kernel here")


# speedup vs baseline: 2.4537x; 2.4537x over previous
import jax
import jax.numpy as jnp
from jax.experimental import pallas as pl
from jax.experimental.pallas import tpu as pltpu

NT = 4
NH = 12
D = 768
DH = D // NH
K = 8
NF = 2048
HK = NH * K
NEG = -3.0e38


def _logits_kernel(xq_ref, xf_ref, wq_ref, wk_ref, out_ref):
    feat = xf_ref[0]
    kmat = jnp.dot(feat, wk_ref[...], preferred_element_type=jnp.float32)
    q_rows = [
        jnp.dot(xq_ref[0, t:t + 1, :], wq_ref[t],
                preferred_element_type=jnp.float32)
        for t in range(NT)
    ]
    qm = jnp.concatenate(q_rows, axis=0)
    scale = DH ** -0.5
    outs = []
    for h in range(NH):
        qh = qm[:, h * DH:(h + 1) * DH]
        kh = kmat[:, h * DH:(h + 1) * DH]
        lg = jax.lax.dot_general(
            qh, kh, (((1,), (1,)), ((), ())),
            preferred_element_type=jnp.float32) * scale
        outs.append(lg)
    out_ref[0] = jnp.concatenate(outs, axis=0)


def _topk_kernel(lg_ref, ti_ref, tw_ref):
    l = lg_ref[...]
    rows = l.shape[0]
    iota = jax.lax.broadcasted_iota(jnp.int32, l.shape, 1)
    vals, idxs = [], []
    for _ in range(K):
        m = jnp.max(l, axis=1, keepdims=True)
        am = jnp.min(jnp.where(l == m, iota, NF), axis=1, keepdims=True)
        vals.append(m)
        idxs.append(am)
        l = jnp.where(iota == am, NEG, l)
    tv = jnp.concatenate(vals, axis=1)
    ti = jnp.concatenate(idxs, axis=1)
    e = jnp.exp(tv - tv[:, 0:1])
    tvs = e / jnp.sum(e, axis=1, keepdims=True)
    zi = jnp.zeros((rows, 128 - K), jnp.int32)
    zf = jnp.zeros((rows, 128 - K), jnp.float32)
    ti_ref[...] = jnp.concatenate([ti, zi], axis=1)
    tw_ref[...] = jnp.concatenate([tvs, zf], axis=1)


def _expert_kernel(ti_ref, tw_ref, xf_ref, wexp_ref, wv_ref,
                   otok_ref, ofeat_ref):
    t = pl.program_id(1)
    idx_row = ti_ref[0]
    w_row = tw_ref[0]
    iota_n = jax.lax.broadcasted_iota(jnp.int32, (NF, HK), 0)
    g0t = (iota_n == idx_row).astype(jnp.float32)
    gw = g0t * w_row
    feat = xf_ref[0]
    a = jax.lax.dot_general(gw, feat, (((0,), (0,)), ((), ())),
                            preferred_element_type=jnp.float32)
    ih = jax.lax.broadcasted_iota(jnp.int32, (HK, D), 0) // K
    ic = jax.lax.broadcasted_iota(jnp.int32, (HK, D), 1) // DH
    am = jnp.where(ih == ic, a, 0.0)
    p = jnp.dot(am, wexp_ref[0], preferred_element_type=jnp.float32)
    scat = jnp.dot(g0t, p, preferred_element_type=jnp.float32)
    sh = jax.lax.broadcasted_iota(jnp.int32, (NH, HK), 0)
    si = jax.lax.broadcasted_iota(jnp.int32, (NH, HK), 1) // K
    smat = (sh == si).astype(jnp.float32)
    g = jnp.dot(smat, a, preferred_element_type=jnp.float32)
    mv = jnp.dot(g, wv_ref[...], preferred_element_type=jnp.float32)
    eh = jax.lax.broadcasted_iota(jnp.int32, (NH, D), 0)
    ec = jax.lax.broadcasted_iota(jnp.int32, (NH, D), 1) // DH
    att = jnp.sum(jnp.where(eh == ec, mv, 0.0), axis=0, keepdims=True)
    token = jnp.dot(att, wexp_ref[0], preferred_element_type=jnp.float32)
    oh_t = (jax.lax.broadcasted_iota(jnp.int32, (NT, 1), 0) == t
            ).astype(jnp.float32)
    tok_add = oh_t * token

    @pl.when(t == 0)
    def _init():
        otok_ref[0] = tok_add
        ofeat_ref[0] = scat

    @pl.when(t > 0)
    def _acc():
        otok_ref[0] = otok_ref[0] + tok_add
        ofeat_ref[0] = ofeat_ref[0] + scat


def kernel(x, Wq, Wkv, Wexp):
    B = x.shape[0]
    xq = x[:, :NT, :]
    xf = x[:, NT:, :]
    wk = Wkv[:, :D]
    wv = Wkv[:, D:]

    logits = pl.pallas_call(
        _logits_kernel,
        grid=(B,),
        in_specs=[
            pl.BlockSpec((1, NT, D), lambda b: (b, 0, 0)),
            pl.BlockSpec((1, NF, D), lambda b: (b, 0, 0)),
            pl.BlockSpec((NT, D, D), lambda b: (0, 0, 0)),
            pl.BlockSpec((D, D), lambda b: (0, 0)),
        ],
        out_specs=pl.BlockSpec((1, NH * NT, NF), lambda b: (b, 0, 0)),
        out_shape=jax.ShapeDtypeStruct((B, NH * NT, NF), jnp.float32),
    )(xq, xf, Wq, wk)

    rows = B * NH * NT
    lg = logits.reshape(rows, NF)
    ti, tw = pl.pallas_call(
        _topk_kernel,
        out_shape=(
            jax.ShapeDtypeStruct((rows, 128), jnp.int32),
            jax.ShapeDtypeStruct((rows, 128), jnp.float32),
        ),
    )(lg)

    ti8 = ti[:, :K].reshape(B, NH, NT, K).transpose(0, 2, 1, 3)
    ti8 = ti8.reshape(B * NT, 1, HK)
    tw8 = tw[:, :K].reshape(B, NH, NT, K).transpose(0, 2, 1, 3)
    tw8 = tw8.reshape(B * NT, 1, HK)

    otok, ofeat = pl.pallas_call(
        _expert_kernel,
        grid=(B, NT),
        in_specs=[
            pl.BlockSpec((1, 1, HK), lambda b, t: (b * NT + t, 0, 0)),
            pl.BlockSpec((1, 1, HK), lambda b, t: (b * NT + t, 0, 0)),
            pl.BlockSpec((1, NF, D), lambda b, t: (b, 0, 0)),
            pl.BlockSpec((1, D, D), lambda b, t: (t, 0, 0)),
            pl.BlockSpec((D, D), lambda b, t: (0, 0)),
        ],
        out_specs=(
            pl.BlockSpec((1, NT, D), lambda b, t: (b, 0, 0)),
            pl.BlockSpec((1, NF, D), lambda b, t: (b, 0, 0)),
        ),
        out_shape=(
            jax.ShapeDtypeStruct((B, NT, D), jnp.float32),
            jax.ShapeDtypeStruct((B, NF, D), jnp.float32),
        ),
        compiler_params=pltpu.CompilerParams(
            dimension_semantics=("arbitrary", "arbitrary")),
    )(ti8, tw8, xf, Wexp, wv)

    return jnp.concatenate([otok, ofeat], axis=1)


# speedup vs baseline: 5.9062x; 2.4070x over previous
import jax
import jax.numpy as jnp
from jax.experimental import pallas as pl
from jax.experimental.pallas import tpu as pltpu

NT = 4
NH = 12
D = 768
DH = D // NH
K = 8
NF = 2048
N = NT + NF
HK = NH * K
S = NT * HK
NEG = -3.0e38


def _logits_kernel(x_ref, wq_ref, wk_ref, out_ref):
    feat = x_ref[0, NT:, :]
    kmat = jnp.dot(feat, wk_ref[...], preferred_element_type=jnp.float32)
    q_rows = [
        jnp.dot(x_ref[0, t:t + 1, :], wq_ref[t],
                preferred_element_type=jnp.float32)
        for t in range(NT)
    ]
    qm = jnp.concatenate(q_rows, axis=0)
    r_i = jax.lax.broadcasted_iota(jnp.int32, (NH * NT, NT), 0) % NT
    t_i = jax.lax.broadcasted_iota(jnp.int32, (NH * NT, NT), 1)
    texp = (r_i == t_i).astype(jnp.float32)
    qex = jnp.dot(texp, qm, preferred_element_type=jnp.float32)
    rh = jax.lax.broadcasted_iota(jnp.int32, (NH * NT, D), 0) // NT
    ch = jax.lax.broadcasted_iota(jnp.int32, (NH * NT, D), 1) // DH
    qmask = jnp.where(rh == ch, qex, 0.0)
    scale = DH ** -0.5
    out_ref[0] = jax.lax.dot_general(
        qmask, kmat, (((1,), (1,)), ((), ())),
        preferred_element_type=jnp.float32) * scale


def _topk_kernel(lg_ref, ti_ref, tw_ref):
    l = lg_ref[...]
    rows = l.shape[0]
    iota = jax.lax.broadcasted_iota(jnp.int32, l.shape, 1)
    vals, idxs = [], []
    for _ in range(K):
        m = jnp.max(l, axis=1, keepdims=True)
        am = jnp.min(jnp.where(l == m, iota, NF), axis=1, keepdims=True)
        vals.append(m)
        idxs.append(am)
        l = jnp.where(iota == am, NEG, l)
    tv = jnp.concatenate(vals, axis=1)
    ti = jnp.concatenate(idxs, axis=1)
    e = jnp.exp(tv - tv[:, 0:1])
    tvs = e / jnp.sum(e, axis=1, keepdims=True)
    zi = jnp.zeros((rows, 128 - K), jnp.int32)
    zf = jnp.zeros((rows, 128 - K), jnp.float32)
    ti_ref[...] = jnp.concatenate([ti, zi], axis=1)
    tw_ref[...] = jnp.concatenate([tvs, zf], axis=1)


def _expert_kernel(ti_ref, tw_ref, x_ref, wexp_ref, wv_ref, out_ref):
    idx_row = ti_ref[0]
    w_row = tw_ref[0]
    iota_n = jax.lax.broadcasted_iota(jnp.int32, (NF, S), 0)
    g0t = (iota_n == idx_row).astype(jnp.float32)
    gw = g0t * w_row
    feat = x_ref[0, NT:, :]
    a = jax.lax.dot_general(gw, feat, (((0,), (0,)), ((), ())),
                            preferred_element_type=jnp.float32)
    ih = (jax.lax.broadcasted_iota(jnp.int32, (S, D), 0) // K) % NH
    ic = jax.lax.broadcasted_iota(jnp.int32, (S, D), 1) // DH
    am = jnp.where(ih == ic, a, 0.0)
    p_rows = [
        jnp.dot(am[t * HK:(t + 1) * HK, :], wexp_ref[t],
                preferred_element_type=jnp.float32)
        for t in range(NT)
    ]
    p = jnp.concatenate(p_rows, axis=0)
    scat = jnp.dot(g0t, p, preferred_element_type=jnp.float32)
    sr = jax.lax.broadcasted_iota(jnp.int32, (NT * NH, S), 0)
    si = jax.lax.broadcasted_iota(jnp.int32, (NT * NH, S), 1) // K
    smat = (sr == si).astype(jnp.float32)
    g = jnp.dot(smat, a, preferred_element_type=jnp.float32)
    mv = jnp.dot(g, wv_ref[...], preferred_element_type=jnp.float32)
    er = jax.lax.broadcasted_iota(jnp.int32, (NT * NH, D), 0) % NH
    ec = jax.lax.broadcasted_iota(jnp.int32, (NT * NH, D), 1) // DH
    mvm = jnp.where(er == ec, mv, 0.0)
    tr = jax.lax.broadcasted_iota(jnp.int32, (NT, NT * NH), 0)
    tc = jax.lax.broadcasted_iota(jnp.int32, (NT, NT * NH), 1) // NH
    tsel = (tr == tc).astype(jnp.float32)
    att = jnp.dot(tsel, mvm, preferred_element_type=jnp.float32)
    tok_rows = [
        jnp.dot(att[t:t + 1, :], wexp_ref[t],
                preferred_element_type=jnp.float32)
        for t in range(NT)
    ]
    tok = jnp.concatenate(tok_rows, axis=0)
    out_ref[0] = jnp.concatenate([tok, scat], axis=0)


def kernel(x, Wq, Wkv, Wexp):
    B = x.shape[0]
    wk = Wkv[:, :D]
    wv = Wkv[:, D:]

    logits = pl.pallas_call(
        _logits_kernel,
        grid=(B,),
        in_specs=[
            pl.BlockSpec((1, N, D), lambda b: (b, 0, 0)),
            pl.BlockSpec((NT, D, D), lambda b: (0, 0, 0)),
            pl.BlockSpec((D, D), lambda b: (0, 0)),
        ],
        out_specs=pl.BlockSpec((1, NH * NT, NF), lambda b: (b, 0, 0)),
        out_shape=jax.ShapeDtypeStruct((B, NH * NT, NF), jnp.float32),
    )(x, Wq, wk)

    rows = B * NH * NT
    lg = logits.reshape(rows, NF)
    ti, tw = pl.pallas_call(
        _topk_kernel,
        out_shape=(
            jax.ShapeDtypeStruct((rows, 128), jnp.int32),
            jax.ShapeDtypeStruct((rows, 128), jnp.float32),
        ),
    )(lg)

    ti8 = ti[:, :K].reshape(B, NH, NT, K).transpose(0, 2, 1, 3)
    ti8 = ti8.reshape(B, 1, S)
    tw8 = tw[:, :K].reshape(B, NH, NT, K).transpose(0, 2, 1, 3)
    tw8 = tw8.reshape(B, 1, S)

    out = pl.pallas_call(
        _expert_kernel,
        grid=(B,),
        in_specs=[
            pl.BlockSpec((1, 1, S), lambda b: (b, 0, 0)),
            pl.BlockSpec((1, 1, S), lambda b: (b, 0, 0)),
            pl.BlockSpec((1, N, D), lambda b: (b, 0, 0)),
            pl.BlockSpec((NT, D, D), lambda b: (0, 0, 0)),
            pl.BlockSpec((D, D), lambda b: (0, 0)),
        ],
        out_specs=pl.BlockSpec((1, N, D), lambda b: (b, 0, 0)),
        out_shape=jax.ShapeDtypeStruct((B, N, D), jnp.float32),
    )(ti8, tw8, x, Wexp, wv)

    return out
